# baseline (speedup 1.0000x reference)
import dataclasses
import functools

import jax
import jax.numpy as jnp
from jax import lax
from jax.experimental import pallas as pl
from jax.experimental.pallas import tpu as pltpu
from jax.experimental.pallas import tpu_sc as plsc

N = 10000
E = 320000
D = 128
NC = 2
NS = 16
NW = NC * NS
CHUNK = 128
GROUP = 8
CPT = 80
NG = CPT // GROUP
EPT = CHUNK * CPT
EPAD = NW * EPT
NPAD = 10112
ROWS_PT = NPAD // NS
DH = 16

_mesh = plsc.VectorSubcoreMesh(core_axis_name="c", subcore_axis_name="s")

_cp = pltpu.CompilerParams()
if "needs_layout_passes" in pltpu.CompilerParams.__dataclass_fields__:
    _cp = dataclasses.replace(_cp, needs_layout_passes=False)


@functools.partial(
    pl.kernel,
    out_type=jax.ShapeDtypeStruct((NW, 2, NPAD), jnp.float32),
    mesh=_mesh,
    compiler_params=_cp,
    scratch_types=[
        pltpu.VMEM((GROUP, CHUNK), jnp.int32),
        pltpu.VMEM((GROUP, CHUNK), jnp.int32),
        pltpu.VMEM((NPAD,), jnp.float32),
        pltpu.VMEM((NPAD,), jnp.float32),
    ],
)
def _sc_degrees(srcs_hbm, dsts_hbm, out_hbm, sidx, didx, hout, hin):
    c = lax.axis_index("c")
    s = lax.axis_index("s")
    wid = s * NC + c

    fill0 = jnp.zeros((16,), jnp.float32)

    @pl.loop(0, NPAD // 16)
    def _(i):
        hout[pl.ds(i * 16, 16)] = fill0
        hin[pl.ds(i * 16, 16)] = fill0

    ones16 = jnp.ones((16,), jnp.float32)

    @pl.loop(0, NG)
    def _(g):
        goff = pl.multiple_of(g * GROUP, 8)
        pltpu.sync_copy(srcs_hbm.at[wid, pl.ds(goff, GROUP)], sidx)
        pltpu.sync_copy(dsts_hbm.at[wid, pl.ds(goff, GROUP)], didx)
        for k in range(GROUP):
            @pl.loop(0, CHUNK // 16)
            def _(t):
                sv = sidx[k, pl.ds(t * 16, 16)]
                dv = didx[k, pl.ds(t * 16, 16)]
                plsc.addupdate_scatter(hout, [sv], ones16)
                plsc.addupdate_scatter(hin, [dv], ones16)

    pltpu.sync_copy(hout, out_hbm.at[wid, 0])
    pltpu.sync_copy(hin, out_hbm.at[wid, 1])


@functools.partial(
    pl.kernel,
    out_type=jax.ShapeDtypeStruct((NC, NPAD, D), jnp.float32),
    mesh=_mesh,
    scratch_types=[
        pltpu.VMEM_SHARED((NPAD, D), jnp.float32),
        pltpu.VMEM((GROUP, CHUNK), jnp.int32),
        pltpu.VMEM((GROUP, CHUNK), jnp.int32),
        pltpu.VMEM((CHUNK, D), jnp.float32),
        pltpu.VMEM((CHUNK, D), jnp.float32),
        pltpu.SemaphoreType.DMA,
        pltpu.SemaphoreType.DMA,
    ],
)
def _sc_aggregate(y_hbm, srcs_hbm, dsts_hbm, out_hbm, acc, sidx, didx,
                  bufa, bufb, sema, semb):
    c = lax.axis_index("c")
    s = lax.axis_index("s")
    wid = s * NC + c

    fill0 = jnp.zeros((16,), jnp.float32)

    @pl.loop(0, CHUNK)
    def _(i):
        for j in range(D // 16):
            bufb[i, pl.ds(j * 16, 16)] = fill0

    @pl.loop(0, ROWS_PT // 8)
    def _(i):
        off = pl.multiple_of(s * ROWS_PT + i * 8, 8)
        pltpu.sync_copy(bufb.at[pl.ds(0, 8)], acc.at[pl.ds(off, 8)])

    plsc.subcore_barrier()

    @pl.loop(0, NG)
    def _(g):
        goff = pl.multiple_of(g * GROUP, 8)
        pltpu.sync_copy(srcs_hbm.at[wid, pl.ds(goff, GROUP)], sidx)
        pltpu.sync_copy(dsts_hbm.at[wid, pl.ds(goff, GROUP)], didx)
        pltpu.async_copy(y_hbm.at[sidx.at[0]], bufa, sema)
        for k in range(GROUP - 1):
            cbuf, csem = (bufa, sema) if k % 2 == 0 else (bufb, semb)
            nbuf, nsem = (bufb, semb) if k % 2 == 0 else (bufa, sema)
            pltpu.async_copy(y_hbm.at[sidx.at[k + 1]], nbuf, nsem)
            pltpu.make_async_copy(y_hbm.at[sidx.at[k]], cbuf, csem).wait()
            pltpu.sync_copy(cbuf, acc.at[didx.at[k]], add=True)
        lbuf, lsem = (bufa, sema) if (GROUP - 1) % 2 == 0 else (bufb, semb)
        pltpu.make_async_copy(y_hbm.at[sidx.at[GROUP - 1]], lbuf, lsem).wait()
        pltpu.sync_copy(lbuf, acc.at[didx.at[GROUP - 1]], add=True)

    plsc.subcore_barrier()

    r0 = s * ROWS_PT
    pltpu.sync_copy(acc.at[pl.ds(r0, ROWS_PT)],
                    out_hbm.at[c, pl.ds(r0, ROWS_PT)])


def _matmul(x, w):
    return lax.dot_general(x, w, (((1,), (0,)), ((), ())),
                           precision=lax.Precision.HIGHEST)


def _tc_first_body(x_ref, w_ref, degs_ref, y_ref, nsrc_ref, ndst_ref):
    degs = degs_ref[...]
    deg = jnp.sum(degs, axis=0)
    norm = jnp.transpose(jax.lax.rsqrt(jnp.clip(deg, 1.0, None)))
    nsrc = norm[:, 0:1]
    ndst = norm[:, 1:2]
    nsrc_ref[...] = nsrc
    ndst_ref[...] = ndst
    y = _matmul(x_ref[...], w_ref[...]) * nsrc[:N]
    y_ref[...] = jnp.concatenate(
        [y, jnp.zeros((NPAD - N, D), jnp.float32)], axis=0)


def _tc_first(x, w, degs):
    return pl.pallas_call(
        _tc_first_body,
        out_shape=(jax.ShapeDtypeStruct((NPAD, D), jnp.float32),
                   jax.ShapeDtypeStruct((NPAD, 1), jnp.float32),
                   jax.ShapeDtypeStruct((NPAD, 1), jnp.float32)),
    )(x, w, degs)


def _tc_mid_body(parts_ref, ndst_ref, b_ref, w_ref, nsrc_ref, y_ref):
    agg = parts_ref[0] + parts_ref[1]
    h = jax.nn.relu(agg * ndst_ref[...] + b_ref[...])
    y = _matmul(h, w_ref[...]) * nsrc_ref[...]
    rows = lax.broadcasted_iota(jnp.int32, (NPAD, D), 0)
    y_ref[...] = jnp.where(rows < N, y, 0.0)


def _tc_mid(parts, ndst, b, w, nsrc):
    return pl.pallas_call(
        _tc_mid_body,
        out_shape=jax.ShapeDtypeStruct((NPAD, D), jnp.float32),
    )(parts, ndst, b, w, nsrc)


def _tc_final_body(parts_ref, ndst_ref, b_ref, out_ref):
    agg = parts_ref[0, pl.ds(0, N)] + parts_ref[1, pl.ds(0, N)]
    out_ref[...] = agg * ndst_ref[pl.ds(0, N)] + b_ref[...]


def _tc_final(parts, ndst, b):
    return pl.pallas_call(
        _tc_final_body,
        out_shape=jax.ShapeDtypeStruct((N, D), jnp.float32),
    )(parts, ndst, b)


def kernel(features, edge_index, W0, b0, W1, b1, W2, b2):
    src = edge_index[0]
    dst = edge_index[1]
    pad = jnp.full((EPAD - E,), N, jnp.int32)
    srcs = jnp.concatenate([src, pad]).reshape(NW, CPT, CHUNK)
    dsts = jnp.concatenate([dst, pad]).reshape(NW, CPT, CHUNK)

    degs = _sc_degrees(srcs, dsts)

    y, nsrc, ndst = _tc_first(features, W0, degs)
    parts = _sc_aggregate(y, srcs, dsts)

    y = _tc_mid(parts, ndst, b0.reshape(1, D), W1, nsrc)
    parts = _sc_aggregate(y, srcs, dsts)

    y = _tc_mid(parts, ndst, b1.reshape(1, D), W2, nsrc)
    parts = _sc_aggregate(y, srcs, dsts)

    return _tc_final(parts, ndst, b2.reshape(1, D))


# speedup vs baseline: 1.0288x; 1.0288x over previous
import dataclasses
import functools

import jax
import jax.numpy as jnp
from jax import lax
from jax.experimental import pallas as pl
from jax.experimental.pallas import tpu as pltpu
from jax.experimental.pallas import tpu_sc as plsc

N = 10000
E = 320000
D = 128
NC = 2
NS = 16
NW = NC * NS
CHUNK = 128
GROUP = 16
CPT = 80
NG = CPT // GROUP
EPT = CHUNK * CPT
EPAD = NW * EPT
NPAD = 10112
ROWS_PT = NPAD // NS
DH = 16

_mesh = plsc.VectorSubcoreMesh(core_axis_name="c", subcore_axis_name="s")

_cp = pltpu.CompilerParams()
if "needs_layout_passes" in pltpu.CompilerParams.__dataclass_fields__:
    _cp = dataclasses.replace(_cp, needs_layout_passes=False)


@functools.partial(
    pl.kernel,
    out_type=jax.ShapeDtypeStruct((NW, 2, NPAD), jnp.float32),
    mesh=_mesh,
    compiler_params=_cp,
    scratch_types=[
        pltpu.VMEM((GROUP, CHUNK), jnp.int32),
        pltpu.VMEM((GROUP, CHUNK), jnp.int32),
        pltpu.VMEM((NPAD,), jnp.float32),
        pltpu.VMEM((NPAD,), jnp.float32),
    ],
)
def _sc_degrees(srcs_hbm, dsts_hbm, out_hbm, sidx, didx, hout, hin):
    c = lax.axis_index("c")
    s = lax.axis_index("s")
    wid = s * NC + c

    fill0 = jnp.zeros((16,), jnp.float32)

    @pl.loop(0, NPAD // 16)
    def _(i):
        hout[pl.ds(i * 16, 16)] = fill0
        hin[pl.ds(i * 16, 16)] = fill0

    ones16 = jnp.ones((16,), jnp.float32)

    @pl.loop(0, NG)
    def _(g):
        goff = pl.multiple_of(g * GROUP, 8)
        pltpu.sync_copy(srcs_hbm.at[wid, pl.ds(goff, GROUP)], sidx)
        pltpu.sync_copy(dsts_hbm.at[wid, pl.ds(goff, GROUP)], didx)
        for k in range(GROUP):
            @pl.loop(0, CHUNK // 16)
            def _(t):
                sv = sidx[k, pl.ds(t * 16, 16)]
                dv = didx[k, pl.ds(t * 16, 16)]
                plsc.addupdate_scatter(hout, [sv], ones16)
                plsc.addupdate_scatter(hin, [dv], ones16)

    pltpu.sync_copy(hout, out_hbm.at[wid, 0])
    pltpu.sync_copy(hin, out_hbm.at[wid, 1])


@functools.partial(
    pl.kernel,
    out_type=jax.ShapeDtypeStruct((NC, NPAD, D), jnp.float32),
    mesh=_mesh,
    scratch_types=[
        pltpu.VMEM_SHARED((NPAD, D), jnp.float32),
        pltpu.VMEM((GROUP, CHUNK), jnp.int32),
        pltpu.VMEM((GROUP, CHUNK), jnp.int32),
        pltpu.VMEM((CHUNK, D), jnp.float32),
        pltpu.VMEM((CHUNK, D), jnp.float32),
        pltpu.SemaphoreType.DMA,
        pltpu.SemaphoreType.DMA,
    ],
)
def _sc_aggregate(y_hbm, srcs_hbm, dsts_hbm, out_hbm, acc, sidx, didx,
                  bufa, bufb, sema, semb):
    c = lax.axis_index("c")
    s = lax.axis_index("s")
    wid = s * NC + c

    fill0 = jnp.zeros((16,), jnp.float32)

    @pl.loop(0, CHUNK)
    def _(i):
        for j in range(D // 16):
            bufb[i, pl.ds(j * 16, 16)] = fill0

    @pl.loop(0, ROWS_PT // 8)
    def _(i):
        off = pl.multiple_of(s * ROWS_PT + i * 8, 8)
        pltpu.sync_copy(bufb.at[pl.ds(0, 8)], acc.at[pl.ds(off, 8)])

    plsc.subcore_barrier()

    @pl.loop(0, NG)
    def _(g):
        goff = pl.multiple_of(g * GROUP, 8)
        pltpu.sync_copy(srcs_hbm.at[wid, pl.ds(goff, GROUP)], sidx)
        pltpu.sync_copy(dsts_hbm.at[wid, pl.ds(goff, GROUP)], didx)
        pltpu.async_copy(y_hbm.at[sidx.at[0]], bufa, sema)
        for k in range(GROUP - 1):
            cbuf, csem = (bufa, sema) if k % 2 == 0 else (bufb, semb)
            nbuf, nsem = (bufb, semb) if k % 2 == 0 else (bufa, sema)
            pltpu.async_copy(y_hbm.at[sidx.at[k + 1]], nbuf, nsem)
            pltpu.make_async_copy(y_hbm.at[sidx.at[k]], cbuf, csem).wait()
            pltpu.sync_copy(cbuf, acc.at[didx.at[k]], add=True)
        lbuf, lsem = (bufa, sema) if (GROUP - 1) % 2 == 0 else (bufb, semb)
        pltpu.make_async_copy(y_hbm.at[sidx.at[GROUP - 1]], lbuf, lsem).wait()
        pltpu.sync_copy(lbuf, acc.at[didx.at[GROUP - 1]], add=True)

    plsc.subcore_barrier()

    r0 = s * ROWS_PT
    pltpu.sync_copy(acc.at[pl.ds(r0, ROWS_PT)],
                    out_hbm.at[c, pl.ds(r0, ROWS_PT)])


def _matmul(x, w):
    return lax.dot_general(x, w, (((1,), (0,)), ((), ())),
                           precision=lax.Precision.HIGHEST)


def _tc_first_body(x_ref, w_ref, degs_ref, y_ref, nsrc_ref, ndst_ref):
    degs = degs_ref[...]
    deg = jnp.sum(degs, axis=0)
    norm = jnp.transpose(jax.lax.rsqrt(jnp.clip(deg, 1.0, None)))
    nsrc = norm[:, 0:1]
    ndst = norm[:, 1:2]
    nsrc_ref[...] = nsrc
    ndst_ref[...] = ndst
    y = _matmul(x_ref[...], w_ref[...]) * nsrc[:N]
    y_ref[...] = jnp.concatenate(
        [y, jnp.zeros((NPAD - N, D), jnp.float32)], axis=0)


def _tc_first(x, w, degs):
    return pl.pallas_call(
        _tc_first_body,
        out_shape=(jax.ShapeDtypeStruct((NPAD, D), jnp.float32),
                   jax.ShapeDtypeStruct((NPAD, 1), jnp.float32),
                   jax.ShapeDtypeStruct((NPAD, 1), jnp.float32)),
    )(x, w, degs)


def _tc_mid_body(parts_ref, ndst_ref, b_ref, w_ref, nsrc_ref, y_ref):
    agg = parts_ref[0] + parts_ref[1]
    h = jax.nn.relu(agg * ndst_ref[...] + b_ref[...])
    y = _matmul(h, w_ref[...]) * nsrc_ref[...]
    rows = lax.broadcasted_iota(jnp.int32, (NPAD, D), 0)
    y_ref[...] = jnp.where(rows < N, y, 0.0)


def _tc_mid(parts, ndst, b, w, nsrc):
    return pl.pallas_call(
        _tc_mid_body,
        out_shape=jax.ShapeDtypeStruct((NPAD, D), jnp.float32),
    )(parts, ndst, b, w, nsrc)


def _tc_final_body(parts_ref, ndst_ref, b_ref, out_ref):
    agg = parts_ref[0, pl.ds(0, N)] + parts_ref[1, pl.ds(0, N)]
    out_ref[...] = agg * ndst_ref[pl.ds(0, N)] + b_ref[...]


def _tc_final(parts, ndst, b):
    return pl.pallas_call(
        _tc_final_body,
        out_shape=jax.ShapeDtypeStruct((N, D), jnp.float32),
    )(parts, ndst, b)


def kernel(features, edge_index, W0, b0, W1, b1, W2, b2):
    src = edge_index[0]
    dst = edge_index[1]
    pad = jnp.full((EPAD - E,), N, jnp.int32)
    srcs = jnp.concatenate([src, pad]).reshape(NW, CPT, CHUNK)
    dsts = jnp.concatenate([dst, pad]).reshape(NW, CPT, CHUNK)

    degs = _sc_degrees(srcs, dsts)

    y, nsrc, ndst = _tc_first(features, W0, degs)
    parts = _sc_aggregate(y, srcs, dsts)

    y = _tc_mid(parts, ndst, b0.reshape(1, D), W1, nsrc)
    parts = _sc_aggregate(y, srcs, dsts)

    y = _tc_mid(parts, ndst, b1.reshape(1, D), W2, nsrc)
    parts = _sc_aggregate(y, srcs, dsts)

    return _tc_final(parts, ndst, b2.reshape(1, D))


# speedup vs baseline: 1.0430x; 1.0138x over previous
import dataclasses
import functools

import jax
import jax.numpy as jnp
from jax import lax
from jax.experimental import pallas as pl
from jax.experimental.pallas import tpu as pltpu
from jax.experimental.pallas import tpu_sc as plsc

N = 10000
E = 320000
D = 128
NC = 2
NS = 16
NW = NC * NS
CHUNK = 128
GROUP = 16
CPT = 80
NG = CPT // GROUP
EPT = CHUNK * CPT
EPAD = NW * EPT
NPAD = 10112
ROWS_PT = NPAD // NS
DH = 16

_mesh = plsc.VectorSubcoreMesh(core_axis_name="c", subcore_axis_name="s")

_cp = pltpu.CompilerParams()
if "needs_layout_passes" in pltpu.CompilerParams.__dataclass_fields__:
    _cp = dataclasses.replace(_cp, needs_layout_passes=False)


@functools.partial(
    pl.kernel,
    out_type=jax.ShapeDtypeStruct((NW, 2, NPAD), jnp.float32),
    mesh=_mesh,
    compiler_params=_cp,
    scratch_types=[
        pltpu.VMEM((GROUP, CHUNK), jnp.int32),
        pltpu.VMEM((GROUP, CHUNK), jnp.int32),
        pltpu.VMEM((NPAD,), jnp.float32),
        pltpu.VMEM((NPAD,), jnp.float32),
    ],
)
def _sc_degrees(srcs_hbm, dsts_hbm, out_hbm, sidx, didx, hout, hin):
    c = lax.axis_index("c")
    s = lax.axis_index("s")
    wid = s * NC + c

    fill0 = jnp.zeros((16,), jnp.float32)

    @pl.loop(0, NPAD // 16)
    def _(i):
        hout[pl.ds(i * 16, 16)] = fill0
        hin[pl.ds(i * 16, 16)] = fill0

    ones16 = jnp.ones((16,), jnp.float32)

    @pl.loop(0, NG)
    def _(g):
        goff = pl.multiple_of(g * GROUP, 8)
        pltpu.sync_copy(srcs_hbm.at[wid, pl.ds(goff, GROUP)], sidx)
        pltpu.sync_copy(dsts_hbm.at[wid, pl.ds(goff, GROUP)], didx)
        for k in range(GROUP):
            @pl.loop(0, CHUNK // 16)
            def _(t):
                sv = sidx[k, pl.ds(t * 16, 16)]
                dv = didx[k, pl.ds(t * 16, 16)]
                plsc.addupdate_scatter(hout, [sv], ones16)
                plsc.addupdate_scatter(hin, [dv], ones16)

    pltpu.sync_copy(hout, out_hbm.at[wid, 0])
    pltpu.sync_copy(hin, out_hbm.at[wid, 1])


@functools.partial(
    pl.kernel,
    out_type=jax.ShapeDtypeStruct((NC, NPAD, D), jnp.float32),
    mesh=_mesh,
    scratch_types=[
        pltpu.VMEM_SHARED((NPAD, D), jnp.float32),
        pltpu.VMEM((GROUP, CHUNK), jnp.int32),
        pltpu.VMEM((GROUP, CHUNK), jnp.int32),
        pltpu.VMEM((CHUNK, D), jnp.float32),
        pltpu.VMEM((CHUNK, D), jnp.float32),
        pltpu.SemaphoreType.DMA,
        pltpu.SemaphoreType.DMA,
    ],
)
def _sc_aggregate(y_hbm, srcs_hbm, dsts_hbm, out_hbm, acc, sidx, didx,
                  bufa, bufb, sema, semb):
    c = lax.axis_index("c")
    s = lax.axis_index("s")
    wid = s * NC + c

    fill0 = jnp.zeros((16,), jnp.float32)

    @pl.loop(0, CHUNK)
    def _(i):
        for j in range(D // 16):
            bufb[i, pl.ds(j * 16, 16)] = fill0

    @pl.loop(0, ROWS_PT // 8)
    def _(i):
        off = pl.multiple_of(s * ROWS_PT + i * 8, 8)
        pltpu.sync_copy(bufb.at[pl.ds(0, 8)], acc.at[pl.ds(off, 8)])

    plsc.subcore_barrier()

    @pl.loop(0, NG)
    def _(g):
        goff = pl.multiple_of(g * GROUP, 8)
        pltpu.sync_copy(srcs_hbm.at[wid, pl.ds(goff, GROUP)], sidx)
        pltpu.async_copy(y_hbm.at[sidx.at[0]], bufa, sema)
        pltpu.sync_copy(dsts_hbm.at[wid, pl.ds(goff, GROUP)], didx)
        for k in range(GROUP - 1):
            cbuf, csem = (bufa, sema) if k % 2 == 0 else (bufb, semb)
            nbuf, nsem = (bufb, semb) if k % 2 == 0 else (bufa, sema)
            pltpu.async_copy(y_hbm.at[sidx.at[k + 1]], nbuf, nsem)
            pltpu.make_async_copy(y_hbm.at[sidx.at[k]], cbuf, csem).wait()
            pltpu.sync_copy(cbuf, acc.at[didx.at[k]], add=True)
        lbuf, lsem = (bufa, sema) if (GROUP - 1) % 2 == 0 else (bufb, semb)
        pltpu.make_async_copy(y_hbm.at[sidx.at[GROUP - 1]], lbuf, lsem).wait()
        pltpu.sync_copy(lbuf, acc.at[didx.at[GROUP - 1]], add=True)

    plsc.subcore_barrier()

    r0 = s * ROWS_PT
    pltpu.sync_copy(acc.at[pl.ds(r0, ROWS_PT)],
                    out_hbm.at[c, pl.ds(r0, ROWS_PT)])


def _matmul(x, w):
    return lax.dot_general(x, w, (((1,), (0,)), ((), ())),
                           precision=lax.Precision.HIGHEST)


def _tc_first_body(x_ref, w_ref, degs_ref, y_ref, nsrc_ref, ndst_ref):
    degs = degs_ref[...]
    deg = jnp.sum(degs, axis=0)
    norm = jnp.transpose(jax.lax.rsqrt(jnp.clip(deg, 1.0, None)))
    nsrc = norm[:, 0:1]
    ndst = norm[:, 1:2]
    nsrc_ref[...] = nsrc
    ndst_ref[...] = ndst
    y = _matmul(x_ref[...], w_ref[...]) * nsrc[:N]
    y_ref[...] = jnp.concatenate(
        [y, jnp.zeros((NPAD - N, D), jnp.float32)], axis=0)


def _tc_first(x, w, degs):
    return pl.pallas_call(
        _tc_first_body,
        out_shape=(jax.ShapeDtypeStruct((NPAD, D), jnp.float32),
                   jax.ShapeDtypeStruct((NPAD, 1), jnp.float32),
                   jax.ShapeDtypeStruct((NPAD, 1), jnp.float32)),
    )(x, w, degs)


def _tc_mid_body(parts_ref, ndst_ref, b_ref, w_ref, nsrc_ref, y_ref):
    agg = parts_ref[0] + parts_ref[1]
    h = jax.nn.relu(agg * ndst_ref[...] + b_ref[...])
    y = _matmul(h, w_ref[...]) * nsrc_ref[...]
    rows = lax.broadcasted_iota(jnp.int32, (NPAD, D), 0)
    y_ref[...] = jnp.where(rows < N, y, 0.0)


def _tc_mid(parts, ndst, b, w, nsrc):
    return pl.pallas_call(
        _tc_mid_body,
        out_shape=jax.ShapeDtypeStruct((NPAD, D), jnp.float32),
    )(parts, ndst, b, w, nsrc)


def _tc_final_body(parts_ref, ndst_ref, b_ref, out_ref):
    agg = parts_ref[0, pl.ds(0, N)] + parts_ref[1, pl.ds(0, N)]
    out_ref[...] = agg * ndst_ref[pl.ds(0, N)] + b_ref[...]


def _tc_final(parts, ndst, b):
    return pl.pallas_call(
        _tc_final_body,
        out_shape=jax.ShapeDtypeStruct((N, D), jnp.float32),
    )(parts, ndst, b)


def kernel(features, edge_index, W0, b0, W1, b1, W2, b2):
    src = edge_index[0]
    dst = edge_index[1]
    pad = jnp.full((EPAD - E,), N, jnp.int32)
    srcs = jnp.concatenate([src, pad]).reshape(NW, CPT, CHUNK)
    dsts = jnp.concatenate([dst, pad]).reshape(NW, CPT, CHUNK)

    degs = _sc_degrees(srcs, dsts)

    y, nsrc, ndst = _tc_first(features, W0, degs)
    parts = _sc_aggregate(y, srcs, dsts)

    y = _tc_mid(parts, ndst, b0.reshape(1, D), W1, nsrc)
    parts = _sc_aggregate(y, srcs, dsts)

    y = _tc_mid(parts, ndst, b1.reshape(1, D), W2, nsrc)
    parts = _sc_aggregate(y, srcs, dsts)

    return _tc_final(parts, ndst, b2.reshape(1, D))


# speedup vs baseline: 1.0538x; 1.0104x over previous
import dataclasses
import functools

import jax
import jax.numpy as jnp
from jax import lax
from jax.experimental import pallas as pl
from jax.experimental.pallas import tpu as pltpu
from jax.experimental.pallas import tpu_sc as plsc

N = 10000
E = 320000
D = 128
NC = 2
NS = 16
NW = NC * NS
CHUNK = 128
GROUP = 40
CPT = 80
NG = CPT // GROUP
EPT = CHUNK * CPT
EPAD = NW * EPT
NPAD = 10112
ROWS_PT = NPAD // NS
DH = 16

_mesh = plsc.VectorSubcoreMesh(core_axis_name="c", subcore_axis_name="s")

_cp = pltpu.CompilerParams()
if "needs_layout_passes" in pltpu.CompilerParams.__dataclass_fields__:
    _cp = dataclasses.replace(_cp, needs_layout_passes=False)


@functools.partial(
    pl.kernel,
    out_type=jax.ShapeDtypeStruct((NW, 2, NPAD), jnp.float32),
    mesh=_mesh,
    compiler_params=_cp,
    scratch_types=[
        pltpu.VMEM((GROUP, CHUNK), jnp.int32),
        pltpu.VMEM((GROUP, CHUNK), jnp.int32),
        pltpu.VMEM((NPAD,), jnp.float32),
        pltpu.VMEM((NPAD,), jnp.float32),
    ],
)
def _sc_degrees(srcs_hbm, dsts_hbm, out_hbm, sidx, didx, hout, hin):
    c = lax.axis_index("c")
    s = lax.axis_index("s")
    wid = s * NC + c

    fill0 = jnp.zeros((16,), jnp.float32)

    @pl.loop(0, NPAD // 16)
    def _(i):
        hout[pl.ds(i * 16, 16)] = fill0
        hin[pl.ds(i * 16, 16)] = fill0

    ones16 = jnp.ones((16,), jnp.float32)

    @pl.loop(0, NG)
    def _(g):
        goff = pl.multiple_of(g * GROUP, 8)
        pltpu.sync_copy(srcs_hbm.at[wid, pl.ds(goff, GROUP)], sidx)
        pltpu.sync_copy(dsts_hbm.at[wid, pl.ds(goff, GROUP)], didx)
        for k in range(GROUP):
            @pl.loop(0, CHUNK // 16)
            def _(t):
                sv = sidx[k, pl.ds(t * 16, 16)]
                dv = didx[k, pl.ds(t * 16, 16)]
                plsc.addupdate_scatter(hout, [sv], ones16)
                plsc.addupdate_scatter(hin, [dv], ones16)

    pltpu.sync_copy(hout, out_hbm.at[wid, 0])
    pltpu.sync_copy(hin, out_hbm.at[wid, 1])


@functools.partial(
    pl.kernel,
    out_type=jax.ShapeDtypeStruct((NC, NPAD, D), jnp.float32),
    mesh=_mesh,
    scratch_types=[
        pltpu.VMEM_SHARED((NPAD, D), jnp.float32),
        pltpu.VMEM((GROUP, CHUNK), jnp.int32),
        pltpu.VMEM((GROUP, CHUNK), jnp.int32),
        pltpu.VMEM((CHUNK, D), jnp.float32),
        pltpu.VMEM((CHUNK, D), jnp.float32),
        pltpu.SemaphoreType.DMA,
        pltpu.SemaphoreType.DMA,
    ],
)
def _sc_aggregate(y_hbm, srcs_hbm, dsts_hbm, out_hbm, acc, sidx, didx,
                  bufa, bufb, sema, semb):
    c = lax.axis_index("c")
    s = lax.axis_index("s")
    wid = s * NC + c

    fill0 = jnp.zeros((16,), jnp.float32)

    @pl.loop(0, CHUNK)
    def _(i):
        for j in range(D // 16):
            bufb[i, pl.ds(j * 16, 16)] = fill0

    @pl.loop(0, ROWS_PT // 8)
    def _(i):
        off = pl.multiple_of(s * ROWS_PT + i * 8, 8)
        pltpu.sync_copy(bufb.at[pl.ds(0, 8)], acc.at[pl.ds(off, 8)])

    plsc.subcore_barrier()

    @pl.loop(0, NG)
    def _(g):
        goff = pl.multiple_of(g * GROUP, 8)
        pltpu.sync_copy(srcs_hbm.at[wid, pl.ds(goff, GROUP)], sidx)
        pltpu.async_copy(y_hbm.at[sidx.at[0]], bufa, sema)
        pltpu.sync_copy(dsts_hbm.at[wid, pl.ds(goff, GROUP)], didx)
        for k in range(GROUP - 1):
            cbuf, csem = (bufa, sema) if k % 2 == 0 else (bufb, semb)
            nbuf, nsem = (bufb, semb) if k % 2 == 0 else (bufa, sema)
            pltpu.async_copy(y_hbm.at[sidx.at[k + 1]], nbuf, nsem)
            pltpu.make_async_copy(y_hbm.at[sidx.at[k]], cbuf, csem).wait()
            pltpu.sync_copy(cbuf, acc.at[didx.at[k]], add=True)
        lbuf, lsem = (bufa, sema) if (GROUP - 1) % 2 == 0 else (bufb, semb)
        pltpu.make_async_copy(y_hbm.at[sidx.at[GROUP - 1]], lbuf, lsem).wait()
        pltpu.sync_copy(lbuf, acc.at[didx.at[GROUP - 1]], add=True)

    plsc.subcore_barrier()

    r0 = s * ROWS_PT
    pltpu.sync_copy(acc.at[pl.ds(r0, ROWS_PT)],
                    out_hbm.at[c, pl.ds(r0, ROWS_PT)])


def _matmul(x, w):
    return lax.dot_general(x, w, (((1,), (0,)), ((), ())),
                           precision=lax.Precision.HIGHEST)


def _tc_first_body(x_ref, w_ref, degs_ref, y_ref, nsrc_ref, ndst_ref):
    degs = degs_ref[...]
    deg = jnp.sum(degs, axis=0)
    norm = jnp.transpose(jax.lax.rsqrt(jnp.clip(deg, 1.0, None)))
    nsrc = norm[:, 0:1]
    ndst = norm[:, 1:2]
    nsrc_ref[...] = nsrc
    ndst_ref[...] = ndst
    y = _matmul(x_ref[...], w_ref[...]) * nsrc[:N]
    y_ref[...] = jnp.concatenate(
        [y, jnp.zeros((NPAD - N, D), jnp.float32)], axis=0)


def _tc_first(x, w, degs):
    return pl.pallas_call(
        _tc_first_body,
        out_shape=(jax.ShapeDtypeStruct((NPAD, D), jnp.float32),
                   jax.ShapeDtypeStruct((NPAD, 1), jnp.float32),
                   jax.ShapeDtypeStruct((NPAD, 1), jnp.float32)),
    )(x, w, degs)


def _tc_mid_body(parts_ref, ndst_ref, b_ref, w_ref, nsrc_ref, y_ref):
    agg = parts_ref[0] + parts_ref[1]
    h = jax.nn.relu(agg * ndst_ref[...] + b_ref[...])
    y = _matmul(h, w_ref[...]) * nsrc_ref[...]
    rows = lax.broadcasted_iota(jnp.int32, (NPAD, D), 0)
    y_ref[...] = jnp.where(rows < N, y, 0.0)


def _tc_mid(parts, ndst, b, w, nsrc):
    return pl.pallas_call(
        _tc_mid_body,
        out_shape=jax.ShapeDtypeStruct((NPAD, D), jnp.float32),
    )(parts, ndst, b, w, nsrc)


def _tc_final_body(parts_ref, ndst_ref, b_ref, out_ref):
    agg = parts_ref[0, pl.ds(0, N)] + parts_ref[1, pl.ds(0, N)]
    out_ref[...] = agg * ndst_ref[pl.ds(0, N)] + b_ref[...]


def _tc_final(parts, ndst, b):
    return pl.pallas_call(
        _tc_final_body,
        out_shape=jax.ShapeDtypeStruct((N, D), jnp.float32),
    )(parts, ndst, b)


def kernel(features, edge_index, W0, b0, W1, b1, W2, b2):
    src = edge_index[0]
    dst = edge_index[1]
    pad = jnp.full((EPAD - E,), N, jnp.int32)
    srcs = jnp.concatenate([src, pad]).reshape(NW, CPT, CHUNK)
    dsts = jnp.concatenate([dst, pad]).reshape(NW, CPT, CHUNK)

    degs = _sc_degrees(srcs, dsts)

    y, nsrc, ndst = _tc_first(features, W0, degs)
    parts = _sc_aggregate(y, srcs, dsts)

    y = _tc_mid(parts, ndst, b0.reshape(1, D), W1, nsrc)
    parts = _sc_aggregate(y, srcs, dsts)

    y = _tc_mid(parts, ndst, b1.reshape(1, D), W2, nsrc)
    parts = _sc_aggregate(y, srcs, dsts)

    return _tc_final(parts, ndst, b2.reshape(1, D))
